# Initial kernel scaffold; baseline (speedup 1.0000x reference)
import jax
import jax.numpy as jnp
from jax.experimental import pallas as pl


def kernel(x):
    raise NotImplementedError("write your pallas kernel here")


# speedup vs baseline: 60.3776x; 60.3776x over previous
import jax
import jax.numpy as jnp
from jax.experimental import pallas as pl


def _body(x_ref, o_ref):
    o_ref[...] = x_ref[...]
    pd = pl.program_id(1)

    @pl.when(pd == 1)
    def _():
        o_ref[0, :, 64:96, 32:64] = jnp.zeros((32, 32, 32), jnp.float32)

    @pl.when(pd == 2)
    def _():
        o_ref[0, :, 32:64, 64:96] = jnp.zeros((32, 32, 32), jnp.float32)
        o_ref[0, :, 64:96, 0:32] = jnp.zeros((32, 32, 32), jnp.float32)


def kernel(x):
    B = x.shape[0]
    x4 = x.reshape(B, 128, 128, 128)
    out = pl.pallas_call(
        _body,
        grid=(B, 4),
        in_specs=[pl.BlockSpec((1, 32, 128, 128), lambda b, pd: (b, pd, 0, 0))],
        out_specs=pl.BlockSpec((1, 32, 128, 128), lambda b, pd: (b, pd, 0, 0)),
        out_shape=jax.ShapeDtypeStruct((B, 128, 128, 128), jnp.float32),
    )(x4)
    return out.reshape(x.shape)
